# baseline (speedup 1.0000x reference)
import functools

import jax
import jax.numpy as jnp
from jax import lax
from jax.experimental import pallas as pl
from jax.experimental.pallas import tpu as pltpu
from jax.experimental.pallas import tpu_sc as plsc

NC = 2
NS = 16
C = 80
BLK = 16
SINK = 128


def _mesh():
  return plsc.VectorSubcoreMesh(core_axis_name="c", subcore_axis_name="s")


def _fill_2d(ref, nrows, value):
  def outer(i, _):
    def inner(j, _):
      ref[i, pl.ds(j * 16, 16)] = jnp.full((16,), value, jnp.float32)
      return 0
    lax.fori_loop(0, 8, inner, 0)
    return 0
  lax.fori_loop(0, nrows, outer, 0)


def _sc_scatter(src3, dst3, g2d, n, with_gather):
  nt, iters, c_ = src3.shape
  nb = iters // BLK
  assert iters % BLK == 0
  rps = (n // NS) // 8 * 8
  tail = n - NS * rps
  zr = 16
  assert rps % zr == 0 and tail <= zr and zr <= C

  @functools.partial(
      pl.kernel,
      out_type=jax.ShapeDtypeStruct((NC, n, 128), jnp.float32),
      mesh=_mesh(),
      scratch_types=[
          pltpu.VMEM_SHARED((n + SINK, 128), jnp.float32),
          pltpu.VMEM((2, BLK, C), jnp.int32),
          pltpu.VMEM((2, BLK, C), jnp.int32),
          pltpu.VMEM((2, C, 128), jnp.float32),
          pltpu.SemaphoreType.DMA((2,)),
          pltpu.SemaphoreType.DMA((2,)),
      ],
  )
  def k(src_hbm, dst_hbm, g_hbm, out_hbm, agg_sh, sidx2, didx2, rows2, gsem,
        isem):
    c = lax.axis_index("c")
    s = lax.axis_index("s")
    t = c * NS + s

    def load_idx(blk, slot):
      if with_gather:
        pltpu.async_copy(src_hbm.at[t, pl.ds(blk * BLK, BLK)],
                         sidx2.at[slot], isem.at[slot])
      pltpu.async_copy(dst_hbm.at[t, pl.ds(blk * BLK, BLK)],
                       didx2.at[slot], isem.at[slot])

    def wait_idx(slot):
      if with_gather:
        pltpu.make_async_copy(src_hbm.at[t, pl.ds(0, BLK)], sidx2.at[slot],
                              isem.at[slot]).wait()
      pltpu.make_async_copy(dst_hbm.at[t, pl.ds(0, BLK)], didx2.at[slot],
                            isem.at[slot]).wait()

    load_idx(0, 0)
    if nb > 1:
      load_idx(1, 1)

    _fill_2d(rows2.at[0], zr, 0.0)

    def zcopy(kk, _):
      pltpu.sync_copy(rows2.at[0, pl.ds(0, zr)],
                      agg_sh.at[pl.ds(s * rps + kk * zr, zr)])
      return 0
    lax.fori_loop(0, rps // zr, zcopy, 0)

    @pl.when(s == 0)
    def _():
      pltpu.sync_copy(rows2.at[0, pl.ds(0, tail)],
                      agg_sh.at[pl.ds(NS * rps, tail)])

    if not with_gather:
      _fill_2d(rows2.at[0], C, 1.0)
    wait_idx(0)
    plsc.subcore_barrier()

    if with_gather:
      pltpu.async_copy(g_hbm.at[sidx2.at[0, 0]], rows2.at[0], gsem.at[0])

      def outer(bk, _):
        bb = lax.rem(bk, 2)

        def inner(j, _):
          i = bk * BLK + j
          b = lax.rem(i, 2)
          pltpu.make_async_copy(g_hbm.at[sidx2.at[bb, j]], rows2.at[b],
                                gsem.at[b]).wait()

          @pl.when(j + 1 < BLK)
          def _():
            pltpu.async_copy(g_hbm.at[sidx2.at[bb, j + 1]], rows2.at[1 - b],
                             gsem.at[1 - b])

          @pl.when(jnp.logical_and(j == BLK - 1, bk + 1 < nb))
          def _():
            wait_idx(1 - bb)
            pltpu.async_copy(g_hbm.at[sidx2.at[1 - bb, 0]], rows2.at[1 - b],
                             gsem.at[1 - b])

            @pl.when(bk + 2 < nb)
            def _():
              load_idx(bk + 2, bb)

          pltpu.sync_copy(rows2.at[b], agg_sh.at[didx2.at[bb, j]], add=True)
          return 0

        lax.fori_loop(0, BLK, inner, 0)
        return 0

      lax.fori_loop(0, nb, outer, 0)
    else:
      def outer(bk, _):
        bb = lax.rem(bk, 2)

        def inner(j, _):
          @pl.when(jnp.logical_and(j == BLK - 1, bk + 1 < nb))
          def _():
            wait_idx(1 - bb)

            @pl.when(bk + 2 < nb)
            def _():
              load_idx(bk + 2, bb)

          pltpu.sync_copy(rows2.at[0], agg_sh.at[didx2.at[bb, j]], add=True)
          return 0

        lax.fori_loop(0, BLK, inner, 0)
        return 0

      lax.fori_loop(0, nb, outer, 0)

    plsc.subcore_barrier()
    pltpu.sync_copy(agg_sh.at[pl.ds(s * rps, rps)],
                    out_hbm.at[c, pl.ds(s * rps, rps)])

    @pl.when(s == 0)
    def _():
      pltpu.sync_copy(agg_sh.at[pl.ds(NS * rps, tail)],
                      out_hbm.at[c, pl.ds(NS * rps, tail)])

  return k(src3, dst3, g2d)


def _dis_block(degp_ref):
  deg = degp_ref[0][:, :1] + degp_ref[1][:, :1] + 1.0
  return lax.rsqrt(deg)


def _tc_first(x, degp, w, n, r=1000):
  def body(x_ref, degp_ref, w_ref, o_ref):
    dis = _dis_block(degp_ref)
    o_ref[...] = jnp.dot(x_ref[...] * dis, w_ref[...],
                         preferred_element_type=jnp.float32)

  return pl.pallas_call(
      body,
      grid=(n // r,),
      in_specs=[
          pl.BlockSpec((r, 128), lambda i: (i, 0)),
          pl.BlockSpec((NC, r, 128), lambda i: (0, i, 0)),
          pl.BlockSpec((128, 128), lambda i: (0, 0)),
      ],
      out_specs=pl.BlockSpec((r, 128), lambda i: (i, 0)),
      out_shape=jax.ShapeDtypeStruct((n, 128), jnp.float32),
  )(x, degp, w)


def _tc_mid(aggp, g, degp, b, w, n, r=1000):
  def body(aggp_ref, g_ref, degp_ref, b_ref, w_ref, o_ref):
    dis = _dis_block(degp_ref)
    agg = aggp_ref[0] + aggp_ref[1] + g_ref[...]
    h = jnp.maximum(agg * dis + b_ref[...], 0.0)
    o_ref[...] = jnp.dot(h * dis, w_ref[...],
                         preferred_element_type=jnp.float32)

  return pl.pallas_call(
      body,
      grid=(n // r,),
      in_specs=[
          pl.BlockSpec((NC, r, 128), lambda i: (0, i, 0)),
          pl.BlockSpec((r, 128), lambda i: (i, 0)),
          pl.BlockSpec((NC, r, 128), lambda i: (0, i, 0)),
          pl.BlockSpec((1, 128), lambda i: (0, 0)),
          pl.BlockSpec((128, 128), lambda i: (0, 0)),
      ],
      out_specs=pl.BlockSpec((r, 128), lambda i: (i, 0)),
      out_shape=jax.ShapeDtypeStruct((n, 128), jnp.float32),
  )(aggp, g, degp, b, w)


def _tc_final(aggp, g, degp, b, n, r=1000):
  def body(aggp_ref, g_ref, degp_ref, b_ref, o_ref):
    dis = _dis_block(degp_ref)
    agg = aggp_ref[0] + aggp_ref[1] + g_ref[...]
    o_ref[...] = agg * dis + b_ref[...]

  return pl.pallas_call(
      body,
      grid=(n // r,),
      in_specs=[
          pl.BlockSpec((NC, r, 128), lambda i: (0, i, 0)),
          pl.BlockSpec((r, 128), lambda i: (i, 0)),
          pl.BlockSpec((NC, r, 128), lambda i: (0, i, 0)),
          pl.BlockSpec((1, 128), lambda i: (0, 0)),
      ],
      out_specs=pl.BlockSpec((r, 128), lambda i: (i, 0)),
      out_shape=jax.ShapeDtypeStruct((n, 128), jnp.float32),
  )(aggp, g, degp, b)


def kernel(x, edge_index, W1, b1, W2, b2, W3, b3):
  n, d = x.shape
  e = edge_index.shape[1]
  nt = NC * NS
  assert d == 128
  iters = -(-e // (nt * C))
  iters = -(-iters // BLK) * BLK
  pad = nt * iters * C - e

  src_flat = edge_index[0]
  dst_flat = edge_index[1]
  if pad:
    src_flat = jnp.concatenate([src_flat, jnp.zeros((pad,), jnp.int32)])
    sink_dst = n + (jnp.arange(pad, dtype=jnp.int32) % SINK)
    dst_flat = jnp.concatenate([dst_flat, sink_dst])
  src3 = src_flat.reshape(nt, iters, C)
  dst3 = dst_flat.reshape(nt, iters, C)
  b1r = b1.reshape(1, 128)
  b2r = b2.reshape(1, 128)
  b3r = b3.reshape(1, 128)

  degp = _sc_scatter(src3, dst3, x, n, with_gather=False)

  g1 = _tc_first(x, degp, W1, n)
  a1 = _sc_scatter(src3, dst3, g1, n, with_gather=True)
  g2 = _tc_mid(a1, g1, degp, b1r, W2, n)
  a2 = _sc_scatter(src3, dst3, g2, n, with_gather=True)
  g3 = _tc_mid(a2, g2, degp, b2r, W3, n)
  a3 = _sc_scatter(src3, dst3, g3, n, with_gather=True)
  return _tc_final(a3, g3, degp, b3r, n)


# speedup vs baseline: 2.6178x; 2.6178x over previous
import functools

import jax
import jax.numpy as jnp
from jax import lax
from jax.experimental import pallas as pl
from jax.experimental.pallas import tpu as pltpu
from jax.experimental.pallas import tpu_sc as plsc

NC = 2
NS = 16
C = 80
BLK = 16
SINK = 128


def _mesh():
  return plsc.VectorSubcoreMesh(core_axis_name="c", subcore_axis_name="s")


def _fill_2d(ref, nrows, value):
  def outer(i, _):
    def inner(j, _):
      ref[i, pl.ds(j * 16, 16)] = jnp.full((16,), value, jnp.float32)
      return 0
    lax.fori_loop(0, 8, inner, 0)
    return 0
  lax.fori_loop(0, nrows, outer, 0)


def _sc_scatter(src3, dst3, g2d, n, with_gather):
  nt, iters, c_ = src3.shape
  nb = iters // BLK
  assert iters % BLK == 0
  rps = (n // NS) // 8 * 8
  tail = n - NS * rps
  zr = 16
  assert rps % zr == 0 and tail <= zr and zr <= C

  @functools.partial(
      pl.kernel,
      out_type=jax.ShapeDtypeStruct((NC, n, 128), jnp.float32),
      mesh=_mesh(),
      scratch_types=[
          pltpu.VMEM_SHARED((n + SINK, 128), jnp.float32),
          pltpu.VMEM((2, BLK, C), jnp.int32),
          pltpu.VMEM((2, BLK, C), jnp.int32),
          pltpu.VMEM((2, C, 128), jnp.float32),
          pltpu.SemaphoreType.DMA((2,)),
          pltpu.SemaphoreType.DMA((2,)),
      ],
  )
  def k(src_hbm, dst_hbm, g_hbm, out_hbm, agg_sh, sidx2, didx2, rows2, gsem,
        isem):
    c = lax.axis_index("c")
    s = lax.axis_index("s")
    t = c * NS + s

    def load_idx(blk, slot):
      if with_gather:
        pltpu.async_copy(src_hbm.at[t, pl.ds(blk * BLK, BLK)],
                         sidx2.at[slot], isem.at[slot])
      pltpu.async_copy(dst_hbm.at[t, pl.ds(blk * BLK, BLK)],
                       didx2.at[slot], isem.at[slot])

    def wait_idx(slot):
      if with_gather:
        pltpu.make_async_copy(src_hbm.at[t, pl.ds(0, BLK)], sidx2.at[slot],
                              isem.at[slot]).wait()
      pltpu.make_async_copy(dst_hbm.at[t, pl.ds(0, BLK)], didx2.at[slot],
                            isem.at[slot]).wait()

    load_idx(0, 0)
    if nb > 1:
      load_idx(1, 1)

    _fill_2d(rows2.at[0], zr, 0.0)

    def zcopy(kk, _):
      pltpu.sync_copy(rows2.at[0, pl.ds(0, zr)],
                      agg_sh.at[pl.ds(s * rps + kk * zr, zr)])
      return 0
    lax.fori_loop(0, rps // zr, zcopy, 0)

    @pl.when(s == 0)
    def _():
      pltpu.sync_copy(rows2.at[0, pl.ds(0, tail)],
                      agg_sh.at[pl.ds(NS * rps, tail)])

    if not with_gather:
      _fill_2d(rows2.at[0], C, 1.0)
    wait_idx(0)
    plsc.subcore_barrier()

    if with_gather:
      pltpu.async_copy(g_hbm.at[sidx2.at[0, 0]], rows2.at[0], gsem.at[0])

      def outer(bk, _):
        bb = lax.rem(bk, 2)

        def inner(j, _):
          i = bk * BLK + j
          b = lax.rem(i, 2)
          pltpu.make_async_copy(g_hbm.at[sidx2.at[bb, j]], rows2.at[b],
                                gsem.at[b]).wait()

          @pl.when(j + 1 < BLK)
          def _():
            pltpu.async_copy(g_hbm.at[sidx2.at[bb, j + 1]], rows2.at[1 - b],
                             gsem.at[1 - b])

          @pl.when(jnp.logical_and(j == BLK - 1, bk + 1 < nb))
          def _():
            wait_idx(1 - bb)
            pltpu.async_copy(g_hbm.at[sidx2.at[1 - bb, 0]], rows2.at[1 - b],
                             gsem.at[1 - b])

            @pl.when(bk + 2 < nb)
            def _():
              load_idx(bk + 2, bb)

          pltpu.sync_copy(rows2.at[b], agg_sh.at[didx2.at[bb, j]], add=True)
          return 0

        lax.fori_loop(0, BLK, inner, 0)
        return 0

      lax.fori_loop(0, nb, outer, 0)
    else:
      def outer(bk, _):
        bb = lax.rem(bk, 2)

        def inner(j, _):
          @pl.when(jnp.logical_and(j == BLK - 1, bk + 1 < nb))
          def _():
            wait_idx(1 - bb)

            @pl.when(bk + 2 < nb)
            def _():
              load_idx(bk + 2, bb)

          pltpu.sync_copy(rows2.at[0], agg_sh.at[didx2.at[bb, j]], add=True)
          return 0

        lax.fori_loop(0, BLK, inner, 0)
        return 0

      lax.fori_loop(0, nb, outer, 0)

    plsc.subcore_barrier()
    pltpu.sync_copy(agg_sh.at[pl.ds(s * rps, rps)],
                    out_hbm.at[c, pl.ds(s * rps, rps)])

    @pl.when(s == 0)
    def _():
      pltpu.sync_copy(agg_sh.at[pl.ds(NS * rps, tail)],
                      out_hbm.at[c, pl.ds(NS * rps, tail)])

  return k(src3, dst3, g2d)


def _dis_block(degp_ref):
  deg = degp_ref[0][:, :1] + degp_ref[1][:, :1] + 1.0
  return lax.rsqrt(deg)


def _tc_first(x, degp, w, n, r=1000):
  def body(x_ref, degp_ref, w_ref, o_ref):
    dis = _dis_block(degp_ref)
    o_ref[...] = jnp.dot(x_ref[...] * dis, w_ref[...],
                         preferred_element_type=jnp.float32)

  return pl.pallas_call(
      body,
      grid=(n // r,),
      in_specs=[
          pl.BlockSpec((r, 128), lambda i: (i, 0)),
          pl.BlockSpec((NC, r, 128), lambda i: (0, i, 0)),
          pl.BlockSpec((128, 128), lambda i: (0, 0)),
      ],
      out_specs=pl.BlockSpec((r, 128), lambda i: (i, 0)),
      out_shape=jax.ShapeDtypeStruct((n, 128), jnp.float32),
  )(x, degp, w)


def _tc_mid(aggp, g, degp, b, w, n, r=1000):
  def body(aggp_ref, g_ref, degp_ref, b_ref, w_ref, o_ref):
    dis = _dis_block(degp_ref)
    agg = aggp_ref[0] + aggp_ref[1] + g_ref[...]
    h = jnp.maximum(agg * dis + b_ref[...], 0.0)
    o_ref[...] = jnp.dot(h * dis, w_ref[...],
                         preferred_element_type=jnp.float32)

  return pl.pallas_call(
      body,
      grid=(n // r,),
      in_specs=[
          pl.BlockSpec((NC, r, 128), lambda i: (0, i, 0)),
          pl.BlockSpec((r, 128), lambda i: (i, 0)),
          pl.BlockSpec((NC, r, 128), lambda i: (0, i, 0)),
          pl.BlockSpec((1, 128), lambda i: (0, 0)),
          pl.BlockSpec((128, 128), lambda i: (0, 0)),
      ],
      out_specs=pl.BlockSpec((r, 128), lambda i: (i, 0)),
      out_shape=jax.ShapeDtypeStruct((n, 128), jnp.float32),
  )(aggp, g, degp, b, w)


def _tc_final(aggp, g, degp, b, n, r=1000):
  def body(aggp_ref, g_ref, degp_ref, b_ref, o_ref):
    dis = _dis_block(degp_ref)
    agg = aggp_ref[0] + aggp_ref[1] + g_ref[...]
    o_ref[...] = agg * dis + b_ref[...]

  return pl.pallas_call(
      body,
      grid=(n // r,),
      in_specs=[
          pl.BlockSpec((NC, r, 128), lambda i: (0, i, 0)),
          pl.BlockSpec((r, 128), lambda i: (i, 0)),
          pl.BlockSpec((NC, r, 128), lambda i: (0, i, 0)),
          pl.BlockSpec((1, 128), lambda i: (0, 0)),
      ],
      out_specs=pl.BlockSpec((r, 128), lambda i: (i, 0)),
      out_shape=jax.ShapeDtypeStruct((n, 128), jnp.float32),
  )(aggp, g, degp, b)


def kernel(x, edge_index, W1, b1, W2, b2, W3, b3):
  n, d = x.shape
  e = edge_index.shape[1]
  nt = NC * NS
  assert d == 128
  iters = -(-e // (nt * C))
  iters = -(-iters // BLK) * BLK
  pad = nt * iters * C - e

  src_flat = edge_index[0]
  dst_flat = edge_index[1]
  if pad:
    pad_ar = jnp.arange(pad, dtype=jnp.int32)
    src_flat = jnp.concatenate([src_flat, pad_ar % n])
    dst_flat = jnp.concatenate([dst_flat, n + pad_ar % SINK])
  src3 = src_flat.reshape(nt, iters, C)
  dst3 = dst_flat.reshape(nt, iters, C)
  b1r = b1.reshape(1, 128)
  b2r = b2.reshape(1, 128)
  b3r = b3.reshape(1, 128)

  degp = _sc_scatter(src3, dst3, x, n, with_gather=False)

  g1 = _tc_first(x, degp, W1, n)
  a1 = _sc_scatter(src3, dst3, g1, n, with_gather=True)
  g2 = _tc_mid(a1, g1, degp, b1r, W2, n)
  a2 = _sc_scatter(src3, dst3, g2, n, with_gather=True)
  g3 = _tc_mid(a2, g2, degp, b2r, W3, n)
  a3 = _sc_scatter(src3, dst3, g3, n, with_gather=True)
  return _tc_final(a3, g3, degp, b3r, n)


# speedup vs baseline: 3.4148x; 1.3044x over previous
import functools

import jax
import jax.numpy as jnp
from jax import lax
from jax.experimental import pallas as pl
from jax.experimental.pallas import tpu as pltpu
from jax.experimental.pallas import tpu_sc as plsc

NC = 2
NS = 16
C = 64
BLK = 16
SINK = 128


def _mesh():
  return plsc.VectorSubcoreMesh(core_axis_name="c", subcore_axis_name="s")


def _fill_2d(ref, nrows, value):
  def outer(i, _):
    def inner(j, _):
      ref[i, pl.ds(j * 16, 16)] = jnp.full((16,), value, jnp.float32)
      return 0
    lax.fori_loop(0, 8, inner, 0)
    return 0
  lax.fori_loop(0, nrows, outer, 0)


def _sc_scatter(src3, dst3, g2d, n, with_gather):
  nt, iters, c_ = src3.shape
  nb = iters // BLK
  assert iters % BLK == 0
  rps = (n // NS) // 8 * 8
  tail = n - NS * rps
  zr = 16
  assert rps % zr == 0 and tail <= zr and zr <= C

  @functools.partial(
      pl.kernel,
      out_type=jax.ShapeDtypeStruct((NC, n, 128), jnp.float32),
      mesh=_mesh(),
      scratch_types=[
          pltpu.VMEM_SHARED((n + SINK, 128), jnp.float32),
          pltpu.VMEM((2, BLK, C), jnp.int32),
          pltpu.VMEM((2, BLK, C), jnp.int32),
          pltpu.VMEM((3, C, 128), jnp.float32),
          pltpu.SemaphoreType.DMA((3,)),
          pltpu.SemaphoreType.DMA((2,)),
      ],
  )
  def k(src_hbm, dst_hbm, g_hbm, out_hbm, agg_sh, sidx2, didx2, rows3, gsem,
        isem):
    c = lax.axis_index("c")
    s = lax.axis_index("s")
    t = c * NS + s

    def load_idx(blk, slot):
      if with_gather:
        pltpu.async_copy(src_hbm.at[t, pl.ds(blk * BLK, BLK)],
                         sidx2.at[slot], isem.at[slot])
      pltpu.async_copy(dst_hbm.at[t, pl.ds(blk * BLK, BLK)],
                       didx2.at[slot], isem.at[slot])

    def wait_idx(slot):
      if with_gather:
        pltpu.make_async_copy(src_hbm.at[t, pl.ds(0, BLK)], sidx2.at[slot],
                              isem.at[slot]).wait()
      pltpu.make_async_copy(dst_hbm.at[t, pl.ds(0, BLK)], didx2.at[slot],
                            isem.at[slot]).wait()

    load_idx(0, 0)

    _fill_2d(rows3.at[0], zr, 0.0)

    def zcopy(kk, _):
      pltpu.sync_copy(rows3.at[0, pl.ds(0, zr)],
                      agg_sh.at[pl.ds(s * rps + kk * zr, zr)])
      return 0
    lax.fori_loop(0, rps // zr, zcopy, 0)

    @pl.when(s == 0)
    def _():
      pltpu.sync_copy(rows3.at[0, pl.ds(0, tail)],
                      agg_sh.at[pl.ds(NS * rps, tail)])

    if not with_gather:
      _fill_2d(rows3.at[0], C, 1.0)
    wait_idx(0)
    plsc.subcore_barrier()

    def issue_gather(m):
      slot = lax.rem(m // BLK, 2)
      pltpu.async_copy(g_hbm.at[sidx2.at[slot, lax.rem(m, BLK)]],
                       rows3.at[lax.rem(m, 3)], gsem.at[lax.rem(m, 3)])

    if with_gather:
      issue_gather(0)
      if iters > 1:
        issue_gather(1)

    def body(i, _):
      bk = i // BLK
      j = lax.rem(i, BLK)
      bb = lax.rem(bk, 2)
      b = lax.rem(i, 3)
      if with_gather:
        pltpu.make_async_copy(g_hbm.at[sidx2.at[bb, j]], rows3.at[b],
                              gsem.at[b]).wait()

      @pl.when(jnp.logical_and(j == 0, bk + 1 < nb))
      def _():
        load_idx(bk + 1, 1 - bb)

      @pl.when(jnp.logical_and(j == BLK - 3, bk + 1 < nb))
      def _():
        wait_idx(1 - bb)

      if with_gather:
        @pl.when(i + 2 < iters)
        def _():
          issue_gather(i + 2)
        pltpu.sync_copy(rows3.at[b], agg_sh.at[didx2.at[bb, j]], add=True)
      else:
        pltpu.sync_copy(rows3.at[0], agg_sh.at[didx2.at[bb, j]], add=True)
      return 0

    lax.fori_loop(0, iters, body, 0)

    plsc.subcore_barrier()
    pltpu.sync_copy(agg_sh.at[pl.ds(s * rps, rps)],
                    out_hbm.at[c, pl.ds(s * rps, rps)])

    @pl.when(s == 0)
    def _():
      pltpu.sync_copy(agg_sh.at[pl.ds(NS * rps, tail)],
                      out_hbm.at[c, pl.ds(NS * rps, tail)])

  return k(src3, dst3, g2d)


def _dis_block(degp_ref):
  deg = degp_ref[0][:, :1] + degp_ref[1][:, :1] + 1.0
  return lax.rsqrt(deg)


def _tc_first(x, degp, w, n, r=1000):
  def body(x_ref, degp_ref, w_ref, o_ref):
    dis = _dis_block(degp_ref)
    o_ref[...] = jnp.dot(x_ref[...] * dis, w_ref[...],
                         preferred_element_type=jnp.float32)

  return pl.pallas_call(
      body,
      grid=(n // r,),
      in_specs=[
          pl.BlockSpec((r, 128), lambda i: (i, 0)),
          pl.BlockSpec((NC, r, 128), lambda i: (0, i, 0)),
          pl.BlockSpec((128, 128), lambda i: (0, 0)),
      ],
      out_specs=pl.BlockSpec((r, 128), lambda i: (i, 0)),
      out_shape=jax.ShapeDtypeStruct((n, 128), jnp.float32),
  )(x, degp, w)


def _tc_mid(aggp, g, degp, b, w, n, r=1000):
  def body(aggp_ref, g_ref, degp_ref, b_ref, w_ref, o_ref):
    dis = _dis_block(degp_ref)
    agg = aggp_ref[0] + aggp_ref[1] + g_ref[...]
    h = jnp.maximum(agg * dis + b_ref[...], 0.0)
    o_ref[...] = jnp.dot(h * dis, w_ref[...],
                         preferred_element_type=jnp.float32)

  return pl.pallas_call(
      body,
      grid=(n // r,),
      in_specs=[
          pl.BlockSpec((NC, r, 128), lambda i: (0, i, 0)),
          pl.BlockSpec((r, 128), lambda i: (i, 0)),
          pl.BlockSpec((NC, r, 128), lambda i: (0, i, 0)),
          pl.BlockSpec((1, 128), lambda i: (0, 0)),
          pl.BlockSpec((128, 128), lambda i: (0, 0)),
      ],
      out_specs=pl.BlockSpec((r, 128), lambda i: (i, 0)),
      out_shape=jax.ShapeDtypeStruct((n, 128), jnp.float32),
  )(aggp, g, degp, b, w)


def _tc_final(aggp, g, degp, b, n, r=1000):
  def body(aggp_ref, g_ref, degp_ref, b_ref, o_ref):
    dis = _dis_block(degp_ref)
    agg = aggp_ref[0] + aggp_ref[1] + g_ref[...]
    o_ref[...] = agg * dis + b_ref[...]

  return pl.pallas_call(
      body,
      grid=(n // r,),
      in_specs=[
          pl.BlockSpec((NC, r, 128), lambda i: (0, i, 0)),
          pl.BlockSpec((r, 128), lambda i: (i, 0)),
          pl.BlockSpec((NC, r, 128), lambda i: (0, i, 0)),
          pl.BlockSpec((1, 128), lambda i: (0, 0)),
      ],
      out_specs=pl.BlockSpec((r, 128), lambda i: (i, 0)),
      out_shape=jax.ShapeDtypeStruct((n, 128), jnp.float32),
  )(aggp, g, degp, b)


def kernel(x, edge_index, W1, b1, W2, b2, W3, b3):
  n, d = x.shape
  e = edge_index.shape[1]
  nt = NC * NS
  assert d == 128
  iters = -(-e // (nt * C))
  iters = -(-iters // BLK) * BLK
  pad = nt * iters * C - e

  src_flat = edge_index[0]
  dst_flat = edge_index[1]
  if pad:
    pad_ar = jnp.arange(pad, dtype=jnp.int32)
    src_flat = jnp.concatenate([src_flat, pad_ar % n])
    dst_flat = jnp.concatenate([dst_flat, n + pad_ar % SINK])
  src3 = src_flat.reshape(nt, iters, C)
  dst3 = dst_flat.reshape(nt, iters, C)
  b1r = b1.reshape(1, 128)
  b2r = b2.reshape(1, 128)
  b3r = b3.reshape(1, 128)

  degp = _sc_scatter(src3, dst3, x, n, with_gather=False)

  g1 = _tc_first(x, degp, W1, n)
  a1 = _sc_scatter(src3, dst3, g1, n, with_gather=True)
  g2 = _tc_mid(a1, g1, degp, b1r, W2, n)
  a2 = _sc_scatter(src3, dst3, g2, n, with_gather=True)
  g3 = _tc_mid(a2, g2, degp, b2r, W3, n)
  a3 = _sc_scatter(src3, dst3, g3, n, with_gather=True)
  return _tc_final(a3, g3, degp, b3r, n)


# speedup vs baseline: 3.7640x; 1.1023x over previous
import functools

import jax
import jax.numpy as jnp
from jax import lax
from jax.experimental import pallas as pl
from jax.experimental.pallas import tpu as pltpu
from jax.experimental.pallas import tpu_sc as plsc

NC = 2
NS = 16
C = 64
BLK = 16
RING = 4
SINK = 128


def _mesh():
  return plsc.VectorSubcoreMesh(core_axis_name="c", subcore_axis_name="s")


def _fill_2d(ref, nrows, value):
  def outer(i, _):
    def inner(j, _):
      ref[i, pl.ds(j * 16, 16)] = jnp.full((16,), value, jnp.float32)
      return 0
    lax.fori_loop(0, 8, inner, 0)
    return 0
  lax.fori_loop(0, nrows, outer, 0)


def _sc_scatter(src3, dst3, g2d, n, with_gather):
  nt, iters, c_ = src3.shape
  nb = iters // BLK
  assert iters % BLK == 0
  rps = (n // NS) // 8 * 8
  tail = n - NS * rps
  zr = 16
  assert rps % zr == 0 and tail <= zr and zr <= C

  @functools.partial(
      pl.kernel,
      out_type=jax.ShapeDtypeStruct((NC, n, 128), jnp.float32),
      mesh=_mesh(),
      scratch_types=[
          pltpu.VMEM_SHARED((n + SINK, 128), jnp.float32),
          pltpu.VMEM((2, BLK, C), jnp.int32),
          pltpu.VMEM((2, BLK, C), jnp.int32),
          pltpu.VMEM((RING, C, 128), jnp.float32),
          pltpu.SemaphoreType.DMA((RING,)),
          pltpu.SemaphoreType.DMA((2,)),
      ],
  )
  def k(src_hbm, dst_hbm, g_hbm, out_hbm, agg_sh, sidx2, didx2, rows3, gsem,
        isem):
    c = lax.axis_index("c")
    s = lax.axis_index("s")
    t = c * NS + s

    def load_idx(blk, slot):
      if with_gather:
        pltpu.async_copy(src_hbm.at[t, pl.ds(blk * BLK, BLK)],
                         sidx2.at[slot], isem.at[slot])
      pltpu.async_copy(dst_hbm.at[t, pl.ds(blk * BLK, BLK)],
                       didx2.at[slot], isem.at[slot])

    def wait_idx(slot):
      if with_gather:
        pltpu.make_async_copy(src_hbm.at[t, pl.ds(0, BLK)], sidx2.at[slot],
                              isem.at[slot]).wait()
      pltpu.make_async_copy(dst_hbm.at[t, pl.ds(0, BLK)], didx2.at[slot],
                            isem.at[slot]).wait()

    load_idx(0, 0)

    _fill_2d(rows3.at[0], zr, 0.0)

    def zcopy(kk, _):
      pltpu.sync_copy(rows3.at[0, pl.ds(0, zr)],
                      agg_sh.at[pl.ds(s * rps + kk * zr, zr)])
      return 0
    lax.fori_loop(0, rps // zr, zcopy, 0)

    @pl.when(s == 0)
    def _():
      pltpu.sync_copy(rows3.at[0, pl.ds(0, tail)],
                      agg_sh.at[pl.ds(NS * rps, tail)])

    if not with_gather:
      _fill_2d(rows3.at[0], C, 1.0)
    wait_idx(0)
    plsc.subcore_barrier()

    def issue_gather(m):
      slot = lax.rem(m // BLK, 2)
      pltpu.async_copy(g_hbm.at[sidx2.at[slot, lax.rem(m, BLK)]],
                       rows3.at[lax.rem(m, RING)], gsem.at[lax.rem(m, RING)])

    if with_gather:
      for m in range(min(RING - 1, iters)):
        issue_gather(m)

    def body(i, _):
      bk = i // BLK
      j = lax.rem(i, BLK)
      bb = lax.rem(bk, 2)
      b = lax.rem(i, RING)
      if with_gather:
        pltpu.make_async_copy(g_hbm.at[sidx2.at[bb, j]], rows3.at[b],
                              gsem.at[b]).wait()

      @pl.when(jnp.logical_and(j == 0, bk + 1 < nb))
      def _():
        load_idx(bk + 1, 1 - bb)

      @pl.when(jnp.logical_and(j == BLK - RING, bk + 1 < nb))
      def _():
        wait_idx(1 - bb)

      if with_gather:
        @pl.when(i + RING - 1 < iters)
        def _():
          issue_gather(i + RING - 1)
        pltpu.sync_copy(rows3.at[b], agg_sh.at[didx2.at[bb, j]], add=True)
      else:
        pltpu.sync_copy(rows3.at[0], agg_sh.at[didx2.at[bb, j]], add=True)
      return 0

    lax.fori_loop(0, iters, body, 0)

    plsc.subcore_barrier()
    pltpu.sync_copy(agg_sh.at[pl.ds(s * rps, rps)],
                    out_hbm.at[c, pl.ds(s * rps, rps)])

    @pl.when(s == 0)
    def _():
      pltpu.sync_copy(agg_sh.at[pl.ds(NS * rps, tail)],
                      out_hbm.at[c, pl.ds(NS * rps, tail)])

  return k(src3, dst3, g2d)


def _dis_block(degp_ref):
  deg = degp_ref[0][:, :1] + degp_ref[1][:, :1] + 1.0
  return lax.rsqrt(deg)


def _tc_first(x, degp, w, n, r=1000):
  def body(x_ref, degp_ref, w_ref, o_ref):
    dis = _dis_block(degp_ref)
    o_ref[...] = jnp.dot(x_ref[...] * dis, w_ref[...],
                         preferred_element_type=jnp.float32)

  return pl.pallas_call(
      body,
      grid=(n // r,),
      in_specs=[
          pl.BlockSpec((r, 128), lambda i: (i, 0)),
          pl.BlockSpec((NC, r, 128), lambda i: (0, i, 0)),
          pl.BlockSpec((128, 128), lambda i: (0, 0)),
      ],
      out_specs=pl.BlockSpec((r, 128), lambda i: (i, 0)),
      out_shape=jax.ShapeDtypeStruct((n, 128), jnp.float32),
  )(x, degp, w)


def _tc_mid(aggp, g, degp, b, w, n, r=1000):
  def body(aggp_ref, g_ref, degp_ref, b_ref, w_ref, o_ref):
    dis = _dis_block(degp_ref)
    agg = aggp_ref[0] + aggp_ref[1] + g_ref[...]
    h = jnp.maximum(agg * dis + b_ref[...], 0.0)
    o_ref[...] = jnp.dot(h * dis, w_ref[...],
                         preferred_element_type=jnp.float32)

  return pl.pallas_call(
      body,
      grid=(n // r,),
      in_specs=[
          pl.BlockSpec((NC, r, 128), lambda i: (0, i, 0)),
          pl.BlockSpec((r, 128), lambda i: (i, 0)),
          pl.BlockSpec((NC, r, 128), lambda i: (0, i, 0)),
          pl.BlockSpec((1, 128), lambda i: (0, 0)),
          pl.BlockSpec((128, 128), lambda i: (0, 0)),
      ],
      out_specs=pl.BlockSpec((r, 128), lambda i: (i, 0)),
      out_shape=jax.ShapeDtypeStruct((n, 128), jnp.float32),
  )(aggp, g, degp, b, w)


def _tc_final(aggp, g, degp, b, n, r=1000):
  def body(aggp_ref, g_ref, degp_ref, b_ref, o_ref):
    dis = _dis_block(degp_ref)
    agg = aggp_ref[0] + aggp_ref[1] + g_ref[...]
    o_ref[...] = agg * dis + b_ref[...]

  return pl.pallas_call(
      body,
      grid=(n // r,),
      in_specs=[
          pl.BlockSpec((NC, r, 128), lambda i: (0, i, 0)),
          pl.BlockSpec((r, 128), lambda i: (i, 0)),
          pl.BlockSpec((NC, r, 128), lambda i: (0, i, 0)),
          pl.BlockSpec((1, 128), lambda i: (0, 0)),
      ],
      out_specs=pl.BlockSpec((r, 128), lambda i: (i, 0)),
      out_shape=jax.ShapeDtypeStruct((n, 128), jnp.float32),
  )(aggp, g, degp, b)


def kernel(x, edge_index, W1, b1, W2, b2, W3, b3):
  n, d = x.shape
  e = edge_index.shape[1]
  nt = NC * NS
  assert d == 128
  iters = -(-e // (nt * C))
  iters = -(-iters // BLK) * BLK
  pad = nt * iters * C - e

  src_flat = edge_index[0]
  dst_flat = edge_index[1]
  if pad:
    pad_ar = jnp.arange(pad, dtype=jnp.int32)
    src_flat = jnp.concatenate([src_flat, pad_ar % n])
    dst_flat = jnp.concatenate([dst_flat, n + pad_ar % SINK])
  src3 = src_flat.reshape(nt, iters, C)
  dst3 = dst_flat.reshape(nt, iters, C)
  b1r = b1.reshape(1, 128)
  b2r = b2.reshape(1, 128)
  b3r = b3.reshape(1, 128)

  degp = _sc_scatter(src3, dst3, x, n, with_gather=False)

  g1 = _tc_first(x, degp, W1, n)
  a1 = _sc_scatter(src3, dst3, g1, n, with_gather=True)
  g2 = _tc_mid(a1, g1, degp, b1r, W2, n)
  a2 = _sc_scatter(src3, dst3, g2, n, with_gather=True)
  g3 = _tc_mid(a2, g2, degp, b2r, W3, n)
  a3 = _sc_scatter(src3, dst3, g3, n, with_gather=True)
  return _tc_final(a3, g3, degp, b3r, n)


# speedup vs baseline: 3.8673x; 1.0274x over previous
import functools

import jax
import jax.numpy as jnp
from jax import lax
from jax.experimental import pallas as pl
from jax.experimental.pallas import tpu as pltpu
from jax.experimental.pallas import tpu_sc as plsc

NC = 2
NS = 16
C = 64
BLK = 16
RING = 4
SINK = 128


def _mesh():
  return plsc.VectorSubcoreMesh(core_axis_name="c", subcore_axis_name="s")


def _fill_2d(ref, nrows, value):
  def outer(i, _):
    def inner(j, _):
      ref[i, pl.ds(j * 16, 16)] = jnp.full((16,), value, jnp.float32)
      return 0
    lax.fori_loop(0, 8, inner, 0)
    return 0
  lax.fori_loop(0, nrows, outer, 0)


def _sc_scatter(src3, dst3, g2d, n, with_gather):
  nt, iters, c_ = src3.shape
  nb = iters // BLK
  assert iters % BLK == 0
  rps = (n // NS) // 8 * 8
  tail = n - NS * rps
  zr = 16
  assert rps % zr == 0 and tail <= zr and zr <= C

  @functools.partial(
      pl.kernel,
      out_type=jax.ShapeDtypeStruct((NC, n, 128), jnp.float32),
      mesh=_mesh(),
      scratch_types=[
          pltpu.VMEM_SHARED((n + SINK, 128), jnp.float32),
          pltpu.VMEM((2, BLK, C), jnp.int32),
          pltpu.VMEM((2, BLK, C), jnp.int32),
          pltpu.VMEM((RING, C, 128), jnp.float32),
          pltpu.SemaphoreType.DMA((RING,)),
          pltpu.SemaphoreType.DMA((2,)),
      ],
  )
  def k(src_hbm, dst_hbm, g_hbm, out_hbm, agg_sh, sidx2, didx2, rows3, gsem,
        isem):
    c = lax.axis_index("c")
    s = lax.axis_index("s")
    t = c * NS + s

    def load_idx(blk, slot):
      if with_gather:
        pltpu.async_copy(src_hbm.at[t, pl.ds(blk * BLK, BLK)],
                         sidx2.at[slot], isem.at[slot])
      pltpu.async_copy(dst_hbm.at[t, pl.ds(blk * BLK, BLK)],
                       didx2.at[slot], isem.at[slot])

    def wait_idx(slot):
      if with_gather:
        pltpu.make_async_copy(src_hbm.at[t, pl.ds(0, BLK)], sidx2.at[slot],
                              isem.at[slot]).wait()
      pltpu.make_async_copy(dst_hbm.at[t, pl.ds(0, BLK)], didx2.at[slot],
                            isem.at[slot]).wait()

    load_idx(0, 0)

    _fill_2d(rows3.at[0], zr, 0.0)

    def zcopy(kk, _):
      pltpu.sync_copy(rows3.at[0, pl.ds(0, zr)],
                      agg_sh.at[pl.ds(s * rps + kk * zr, zr)])
      return 0
    lax.fori_loop(0, rps // zr, zcopy, 0)

    @pl.when(s == 0)
    def _():
      pltpu.sync_copy(rows3.at[0, pl.ds(0, tail)],
                      agg_sh.at[pl.ds(NS * rps, tail)])

    if not with_gather:
      _fill_2d(rows3.at[0], C, 1.0)
    wait_idx(0)
    plsc.subcore_barrier()

    def issue_gather(m):
      slot = lax.rem(m // BLK, 2)
      pltpu.async_copy(g_hbm.at[sidx2.at[slot, lax.rem(m, BLK)]],
                       rows3.at[lax.rem(m, RING)], gsem.at[lax.rem(m, RING)])

    if with_gather:
      for m in range(min(RING - 1, iters)):
        issue_gather(m)

    def body(i, _):
      bk = i // BLK
      j = lax.rem(i, BLK)
      bb = lax.rem(bk, 2)
      b = lax.rem(i, RING)
      if with_gather:
        pltpu.make_async_copy(g_hbm.at[sidx2.at[bb, j]], rows3.at[b],
                              gsem.at[b]).wait()

      @pl.when(jnp.logical_and(j == 0, bk + 1 < nb))
      def _():
        load_idx(bk + 1, 1 - bb)

      @pl.when(jnp.logical_and(j == BLK - RING, bk + 1 < nb))
      def _():
        wait_idx(1 - bb)

      if with_gather:
        @pl.when(i + RING - 1 < iters)
        def _():
          issue_gather(i + RING - 1)
        pltpu.sync_copy(rows3.at[b], agg_sh.at[didx2.at[bb, j]], add=True)
      else:
        pltpu.sync_copy(rows3.at[0], agg_sh.at[didx2.at[bb, j]], add=True)
      return 0

    lax.fori_loop(0, iters, body, 0)

    plsc.subcore_barrier()
    pltpu.sync_copy(agg_sh.at[pl.ds(s * rps, rps)],
                    out_hbm.at[c, pl.ds(s * rps, rps)])

    @pl.when(s == 0)
    def _():
      pltpu.sync_copy(agg_sh.at[pl.ds(NS * rps, tail)],
                      out_hbm.at[c, pl.ds(NS * rps, tail)])

  return k(src3, dst3, g2d)


def _tc_first(x, degp, w, n, r=2000):
  def body(x_ref, degp_ref, w_ref, o_ref, dis_ref):
    deg = degp_ref[0][:, :1] + degp_ref[1][:, :1] + 1.0
    dis = lax.rsqrt(deg)
    dis_ref[...] = jnp.broadcast_to(dis, dis_ref.shape)
    o_ref[...] = jnp.dot(x_ref[...] * dis, w_ref[...],
                         preferred_element_type=jnp.float32)

  return pl.pallas_call(
      body,
      grid=(n // r,),
      in_specs=[
          pl.BlockSpec((r, 128), lambda i: (i, 0)),
          pl.BlockSpec((NC, r, 128), lambda i: (0, i, 0)),
          pl.BlockSpec((128, 128), lambda i: (0, 0)),
      ],
      out_specs=[
          pl.BlockSpec((r, 128), lambda i: (i, 0)),
          pl.BlockSpec((r, 128), lambda i: (i, 0)),
      ],
      out_shape=[
          jax.ShapeDtypeStruct((n, 128), jnp.float32),
          jax.ShapeDtypeStruct((n, 128), jnp.float32),
      ],
  )(x, degp, w)


def _tc_mid(aggp, g, dis, b, w, n, r=2000):
  def body(aggp_ref, g_ref, dis_ref, b_ref, w_ref, o_ref):
    dis_blk = dis_ref[...]
    agg = aggp_ref[0] + aggp_ref[1] + g_ref[...]
    h = jnp.maximum(agg * dis_blk + b_ref[...], 0.0)
    o_ref[...] = jnp.dot(h * dis_blk, w_ref[...],
                         preferred_element_type=jnp.float32)

  return pl.pallas_call(
      body,
      grid=(n // r,),
      in_specs=[
          pl.BlockSpec((NC, r, 128), lambda i: (0, i, 0)),
          pl.BlockSpec((r, 128), lambda i: (i, 0)),
          pl.BlockSpec((r, 128), lambda i: (i, 0)),
          pl.BlockSpec((1, 128), lambda i: (0, 0)),
          pl.BlockSpec((128, 128), lambda i: (0, 0)),
      ],
      out_specs=pl.BlockSpec((r, 128), lambda i: (i, 0)),
      out_shape=jax.ShapeDtypeStruct((n, 128), jnp.float32),
  )(aggp, g, dis, b, w)


def _tc_final(aggp, g, dis, b, n, r=2000):
  def body(aggp_ref, g_ref, dis_ref, b_ref, o_ref):
    agg = aggp_ref[0] + aggp_ref[1] + g_ref[...]
    o_ref[...] = agg * dis_ref[...] + b_ref[...]

  return pl.pallas_call(
      body,
      grid=(n // r,),
      in_specs=[
          pl.BlockSpec((NC, r, 128), lambda i: (0, i, 0)),
          pl.BlockSpec((r, 128), lambda i: (i, 0)),
          pl.BlockSpec((r, 128), lambda i: (i, 0)),
          pl.BlockSpec((1, 128), lambda i: (0, 0)),
      ],
      out_specs=pl.BlockSpec((r, 128), lambda i: (i, 0)),
      out_shape=jax.ShapeDtypeStruct((n, 128), jnp.float32),
  )(aggp, g, dis, b)


def kernel(x, edge_index, W1, b1, W2, b2, W3, b3):
  n, d = x.shape
  e = edge_index.shape[1]
  nt = NC * NS
  assert d == 128
  iters = -(-e // (nt * C))
  iters = -(-iters // BLK) * BLK
  pad = nt * iters * C - e

  src_flat = edge_index[0]
  dst_flat = edge_index[1]
  if pad:
    pad_ar = jnp.arange(pad, dtype=jnp.int32)
    src_flat = jnp.concatenate([src_flat, pad_ar % n])
    dst_flat = jnp.concatenate([dst_flat, n + pad_ar % SINK])
  src3 = src_flat.reshape(nt, iters, C)
  dst3 = dst_flat.reshape(nt, iters, C)
  b1r = b1.reshape(1, 128)
  b2r = b2.reshape(1, 128)
  b3r = b3.reshape(1, 128)

  degp = _sc_scatter(src3, dst3, x, n, with_gather=False)

  g1, dis = _tc_first(x, degp, W1, n)
  a1 = _sc_scatter(src3, dst3, g1, n, with_gather=True)
  g2 = _tc_mid(a1, g1, dis, b1r, W2, n)
  a2 = _sc_scatter(src3, dst3, g2, n, with_gather=True)
  g3 = _tc_mid(a2, g2, dis, b2r, W3, n)
  a3 = _sc_scatter(src3, dst3, g3, n, with_gather=True)
  return _tc_final(a3, g3, dis, b3r, n)
